# Initial kernel scaffold; baseline (speedup 1.0000x reference)
import jax
import jax.numpy as jnp
from jax.experimental import pallas as pl


def kernel(x, edge_index, batch, W1, b1, W2, b2):
    raise NotImplementedError("write your pallas kernel here")


# speedup vs baseline: 41.7288x; 41.7288x over previous
import functools

import jax
import jax.numpy as jnp
from jax import lax
from jax.experimental import pallas as pl
from jax.experimental.pallas import tpu as pltpu
from jax.experimental.pallas import tpu_sc as plsc

N = 10000
E = 320000
D = 128
HID = 16
NG = 128

NC, NS = 2, 16
NW = NC * NS
CHUNK = 128
SLICE = 632
NPAD = NS * SLICE
EPT = 10112
C = EPT // CHUNK
E_PAD = NW * EPT

_mesh = plsc.VectorSubcoreMesh(
    core_axis_name="c", subcore_axis_name="s", num_cores=NC, num_subcores=NS)
_sc_params = pltpu.CompilerParams(use_tc_tiling_on_sc=False)


@functools.partial(
    pl.kernel,
    mesh=_mesh,
    compiler_params=_sc_params,
    out_type=jax.ShapeDtypeStruct((NC, NPAD), jnp.float32),
    scratch_types=[
        pltpu.VMEM((C, CHUNK), jnp.int32),
        pltpu.VMEM((CHUNK,), jnp.float32),
        pltpu.VMEM((SLICE,), jnp.float32),
        pltpu.VMEM_SHARED((NPAD,), jnp.float32),
    ],
)
def _deg_kernel(dst_hbm, out_hbm, idx_v, ones_v, stage_v, deg_sp):
    c = lax.axis_index("c")
    s = lax.axis_index("s")
    w = c * NS + s

    @pl.loop(0, CHUNK // 16)
    def _(i):
        ones_v[pl.ds(i * 16, 16)] = jnp.ones((16,), jnp.float32)

    @pl.loop(0, SLICE // 16)
    def _(i):
        stage_v[pl.ds(i * 16, 16)] = jnp.zeros((16,), jnp.float32)

    pltpu.sync_copy(stage_v, deg_sp.at[pl.ds(s * SLICE, SLICE)])
    pltpu.sync_copy(dst_hbm.at[w], idx_v)
    plsc.subcore_barrier()

    @pl.loop(0, C)
    def _(j):
        pltpu.sync_copy(ones_v, deg_sp.at[idx_v.at[j]], add=True)

    plsc.subcore_barrier()
    pltpu.sync_copy(deg_sp.at[pl.ds(s * SLICE, SLICE)], stage_v)
    pltpu.sync_copy(stage_v, out_hbm.at[c, pl.ds(s * SLICE, SLICE)])


@functools.partial(
    pl.kernel,
    mesh=_mesh,
    compiler_params=_sc_params,
    out_type=jax.ShapeDtypeStruct((NC, NPAD, HID), jnp.float32),
    scratch_types=[
        pltpu.VMEM((C, CHUNK), jnp.int32),
        pltpu.VMEM((C, CHUNK), jnp.int32),
        pltpu.VMEM((CHUNK, HID), jnp.float32),
        pltpu.VMEM((SLICE, HID), jnp.float32),
        pltpu.VMEM_SHARED((NPAD, HID), jnp.float32),
        pltpu.SemaphoreType.DMA,
    ],
)
def _msg_kernel(g_hbm, src_hbm, dst_hbm, out_hbm,
                si_v, di_v, rows_v, stage_v, acc_sp, sem):
    c = lax.axis_index("c")
    s = lax.axis_index("s")
    w = c * NS + s

    @pl.loop(0, SLICE)
    def _(i):
        stage_v[i, :] = jnp.zeros((HID,), jnp.float32)

    pltpu.sync_copy(stage_v, acc_sp.at[pl.ds(s * SLICE, SLICE)])
    pltpu.sync_copy(src_hbm.at[w], si_v)
    pltpu.sync_copy(dst_hbm.at[w], di_v)
    plsc.subcore_barrier()

    @pl.loop(0, C)
    def _(j):
        pltpu.async_copy(g_hbm.at[si_v.at[j]], rows_v, sem).wait()
        pltpu.sync_copy(rows_v, acc_sp.at[di_v.at[j]], add=True)

    plsc.subcore_barrier()
    pltpu.sync_copy(acc_sp.at[pl.ds(s * SLICE, SLICE)], stage_v)
    pltpu.sync_copy(stage_v, out_hbm.at[c, pl.ds(s * SLICE, SLICE)])


def _prep_body(x_ref, w1_ref, d0_ref, d1_ref, g_ref, dinv_ref):
    deg = 1.0 + d0_ref[...] + d1_ref[...]
    dinv = lax.rsqrt(deg)
    h = jnp.dot(x_ref[...], w1_ref[...], preferred_element_type=jnp.float32)
    g_ref[...] = h * dinv
    dinv_ref[...] = dinv


def _head_body(p0_ref, p1_ref, g_ref, dinv_ref, batch_ref, b1_ref,
               w2_ref, b2_ref, out_ref):
    acc = p0_ref[...] + p1_ref[...] + g_ref[...]
    out = acc * dinv_ref[...] + b1_ref[...]
    r = jnp.maximum(out, 0.0)
    gid = lax.broadcasted_iota(jnp.int32, (1, NG), 1)
    onehot = (batch_ref[...] == gid).astype(jnp.float32)
    dn = (((0,), (0,)), ((), ()))
    sums = lax.dot_general(onehot, r, dn, preferred_element_type=jnp.float32)
    cnt = lax.dot_general(onehot, jnp.ones((NPAD, 1), jnp.float32), dn,
                          preferred_element_type=jnp.float32)
    pooled = sums / jnp.maximum(cnt, 1.0)
    out_ref[...] = (jnp.dot(pooled, w2_ref[...],
                            preferred_element_type=jnp.float32) + b2_ref[...])


def kernel(x, edge_index, batch, W1, b1, W2, b2):
    src = edge_index[0]
    dst = edge_index[1]
    pad_e = E_PAD - E
    src_p = jnp.concatenate([src, jnp.full((pad_e,), N, jnp.int32)])
    dst_p = jnp.concatenate([dst, jnp.full((pad_e,), N, jnp.int32)])
    src3d = src_p.reshape(NW, C, CHUNK)
    dst3d = dst_p.reshape(NW, C, CHUNK)
    x_p = jnp.pad(x, ((0, NPAD - N), (0, 0)))
    batch_p = jnp.concatenate(
        [batch, jnp.full((NPAD - N,), NG, jnp.int32)]).reshape(NPAD, 1)

    deg_parts = _deg_kernel(dst3d)
    d0 = deg_parts[0].reshape(NPAD, 1)
    d1 = deg_parts[1].reshape(NPAD, 1)

    g, dinv = pl.pallas_call(
        _prep_body,
        out_shape=(jax.ShapeDtypeStruct((NPAD, HID), jnp.float32),
                   jax.ShapeDtypeStruct((NPAD, 1), jnp.float32)),
    )(x_p, W1, d0, d1)

    acc_parts = _msg_kernel(g, src3d, dst3d)

    logits = pl.pallas_call(
        _head_body,
        out_shape=jax.ShapeDtypeStruct((NG, 10), jnp.float32),
    )(acc_parts[0], acc_parts[1], g, dinv, batch_p,
      b1.reshape(1, HID), W2, b2.reshape(1, 10))
    return logits
